# Initial kernel scaffold; baseline (speedup 1.0000x reference)
import functools

import jax
import jax.numpy as jnp
from jax import lax
from jax.experimental import pallas as pl
from jax.experimental.pallas import tpu as pltpu

N = 10000
E = 320000
D = 128
H = 128
P = 128
G = 64

INTERPRET = False

ROW_BLK = 1000
N_BLKS = N // ROW_BLK


def _dinv_body(hist_ref, o_ref):
    deg = jnp.sum(hist_ref[...], axis=0) + 1.0
    o_ref[...] = lax.rsqrt(deg)[:, None]


def _tc_dinv(hist):
    k = hist.shape[0]
    return pl.pallas_call(
        _dinv_body,
        grid=(N_BLKS,),
        in_specs=[pl.BlockSpec((k, ROW_BLK), lambda j: (0, j))],
        out_specs=pl.BlockSpec((ROW_BLK, 1), lambda j: (j, 0)),
        out_shape=jax.ShapeDtypeStruct((N, 1), jnp.float32),
        interpret=INTERPRET,
    )(hist)


def _in_body(x_ref, win_ref, bin_ref, w1_ref, dinv_ref, h_ref, y_ref):
    h = jnp.dot(x_ref[...], win_ref[...],
                preferred_element_type=jnp.float32) + bin_ref[...]
    h_ref[...] = h
    y_ref[...] = dinv_ref[...] * jnp.dot(h, w1_ref[...],
                                         preferred_element_type=jnp.float32)


def _tc_in(x, W_in, b_in, W1, dinv):
    return pl.pallas_call(
        _in_body,
        grid=(N_BLKS,),
        in_specs=[
            pl.BlockSpec((ROW_BLK, D), lambda j: (j, 0)),
            pl.BlockSpec((D, H), lambda j: (0, 0)),
            pl.BlockSpec((1, H), lambda j: (0, 0)),
            pl.BlockSpec((H, H), lambda j: (0, 0)),
            pl.BlockSpec((ROW_BLK, 1), lambda j: (j, 0)),
        ],
        out_specs=[
            pl.BlockSpec((ROW_BLK, H), lambda j: (j, 0)),
            pl.BlockSpec((ROW_BLK, H), lambda j: (j, 0)),
        ],
        out_shape=[
            jax.ShapeDtypeStruct((N, H), jnp.float32),
            jax.ShapeDtypeStruct((N, H), jnp.float32),
        ],
        interpret=INTERPRET,
    )(x, W_in, b_in.reshape(1, H), W1, dinv)


def _layer_body(res_ref, acca_ref, accb_ref, y_ref, dinv_ref, b_ref, wn_ref,
                r_ref, yn_ref):
    dinv = dinv_ref[...]
    g = dinv * (acca_ref[...] + accb_ref[...] + y_ref[...]) + b_ref[...]
    r = jnp.maximum(res_ref[...] + g, 0.0)
    r_ref[...] = r
    yn_ref[...] = dinv * jnp.dot(r, wn_ref[...],
                                 preferred_element_type=jnp.float32)


def _tc_layer(res, acc_a, acc_b, y, dinv, b, W_next):
    return pl.pallas_call(
        _layer_body,
        grid=(N_BLKS,),
        in_specs=[
            pl.BlockSpec((ROW_BLK, H), lambda j: (j, 0)),
            pl.BlockSpec((ROW_BLK, H), lambda j: (j, 0)),
            pl.BlockSpec((ROW_BLK, H), lambda j: (j, 0)),
            pl.BlockSpec((ROW_BLK, H), lambda j: (j, 0)),
            pl.BlockSpec((ROW_BLK, 1), lambda j: (j, 0)),
            pl.BlockSpec((1, H), lambda j: (0, 0)),
            pl.BlockSpec((H, H), lambda j: (0, 0)),
        ],
        out_specs=[
            pl.BlockSpec((ROW_BLK, H), lambda j: (j, 0)),
            pl.BlockSpec((ROW_BLK, H), lambda j: (j, 0)),
        ],
        out_shape=[
            jax.ShapeDtypeStruct((N, H), jnp.float32),
            jax.ShapeDtypeStruct((N, H), jnp.float32),
        ],
        interpret=INTERPRET,
    )(res, acc_a, acc_b, y, dinv, b.reshape(1, H), W_next)


def _layer3_body(res_ref, acca_ref, accb_ref, y_ref, dinv_ref, b_ref, r_ref):
    g = dinv_ref[...] * (acca_ref[...] + accb_ref[...] + y_ref[...]) + b_ref[...]
    r_ref[...] = jnp.maximum(res_ref[...] + g, 0.0)


def _tc_layer3(res, acc_a, acc_b, y, dinv, b):
    return pl.pallas_call(
        _layer3_body,
        grid=(N_BLKS,),
        in_specs=[
            pl.BlockSpec((ROW_BLK, H), lambda j: (j, 0)),
            pl.BlockSpec((ROW_BLK, H), lambda j: (j, 0)),
            pl.BlockSpec((ROW_BLK, H), lambda j: (j, 0)),
            pl.BlockSpec((ROW_BLK, H), lambda j: (j, 0)),
            pl.BlockSpec((ROW_BLK, 1), lambda j: (j, 0)),
            pl.BlockSpec((1, H), lambda j: (0, 0)),
        ],
        out_specs=pl.BlockSpec((ROW_BLK, H), lambda j: (j, 0)),
        out_shape=jax.ShapeDtypeStruct((N, H), jnp.float32),
        interpret=INTERPRET,
    )(res, acc_a, acc_b, y, dinv, b.reshape(1, H))


def _pool_body(batch_ref, r1_ref, r2_ref, r3_ref, ps_ref, cnt_ref):
    j = pl.program_id(0)
    b = batch_ref[0, 0, :]
    gids = lax.broadcasted_iota(jnp.int32, (G, ROW_BLK), 0)
    mask = (b[None, :] == gids).astype(jnp.float32)

    @pl.when(j == 0)
    def _():
        ps_ref[...] = jnp.zeros_like(ps_ref)
        cnt_ref[...] = jnp.zeros_like(cnt_ref)

    ps_ref[:, 0:H] += jnp.dot(mask, r1_ref[...],
                              preferred_element_type=jnp.float32)
    ps_ref[:, H:2 * H] += jnp.dot(mask, r2_ref[...],
                                  preferred_element_type=jnp.float32)
    ps_ref[:, 2 * H:3 * H] += jnp.dot(mask, r3_ref[...],
                                      preferred_element_type=jnp.float32)
    cnt_ref[...] += jnp.sum(mask, axis=1, keepdims=True)


def _tc_pool(batch3, r1, r2, r3):
    return pl.pallas_call(
        _pool_body,
        grid=(N_BLKS,),
        in_specs=[
            pl.BlockSpec((1, 1, ROW_BLK), lambda j: (j, 0, 0)),
            pl.BlockSpec((ROW_BLK, H), lambda j: (j, 0)),
            pl.BlockSpec((ROW_BLK, H), lambda j: (j, 0)),
            pl.BlockSpec((ROW_BLK, H), lambda j: (j, 0)),
        ],
        out_specs=[
            pl.BlockSpec((G, 3 * H), lambda j: (0, 0)),
            pl.BlockSpec((G, 1), lambda j: (0, 0)),
        ],
        out_shape=[
            jax.ShapeDtypeStruct((G, 3 * H), jnp.float32),
            jax.ShapeDtypeStruct((G, 1), jnp.float32),
        ],
        interpret=INTERPRET,
    )(batch3, r1, r2, r3)


def _head_body(ps_ref, cnt_ref, wp1_ref, bp1_ref, wp2_ref, bp2_ref, o_ref):
    pooled = ps_ref[...] / jnp.maximum(cnt_ref[...], 1.0)
    t = jnp.maximum(jnp.dot(pooled, wp1_ref[...],
                            preferred_element_type=jnp.float32) + bp1_ref[...],
                    0.0)
    p = jnp.dot(t, wp2_ref[...],
                preferred_element_type=jnp.float32) + bp2_ref[...]
    nrm = jnp.sqrt(jnp.sum(p * p, axis=1, keepdims=True))
    o_ref[...] = p / jnp.maximum(nrm, 1e-12)


def _tc_head(ps, cnt, Wp1, bp1, Wp2, bp2):
    return pl.pallas_call(
        _head_body,
        out_shape=jax.ShapeDtypeStruct((G, P), jnp.float32),
        interpret=INTERPRET,
    )(ps, cnt, Wp1, bp1.reshape(1, H), Wp2, bp2.reshape(1, P))


def _sc_hist(dst):
    return jax.ops.segment_sum(jnp.ones(E, jnp.float32), dst,
                               num_segments=N).reshape(1, N)


def _sc_scatter(y, src, dst):
    acc = jax.ops.segment_sum(y[src], dst, num_segments=N)
    return acc, jnp.zeros_like(acc)


def kernel(x, edge_index, batch, W_in, b_in, W1, b1, W2, b2, W3, b3,
           Wp1, bp1, Wp2, bp2):
    src, dst = edge_index[0], edge_index[1]

    hist = _sc_hist(dst)
    dinv = _tc_dinv(hist)

    h, y1 = _tc_in(x, W_in, b_in, W1, dinv)
    a1, a1b = _sc_scatter(y1, src, dst)
    r1, y2 = _tc_layer(h, a1, a1b, y1, dinv, b1, W2)
    a2, a2b = _sc_scatter(y2, src, dst)
    r2, y3 = _tc_layer(r1, a2, a2b, y2, dinv, b2, W3)
    a3, a3b = _sc_scatter(y3, src, dst)
    r3 = _tc_layer3(r2, a3, a3b, y3, dinv, b3)

    batch3 = batch.reshape(N_BLKS, 1, ROW_BLK)
    ps, cnt = _tc_pool(batch3, r1, r2, r3)
    return _tc_head(ps, cnt, Wp1, bp1, Wp2, bp2)


# speedup vs baseline: 2.6663x; 2.6663x over previous
import functools

import jax
import jax.numpy as jnp
from jax import lax
from jax.experimental import pallas as pl
from jax.experimental.pallas import tpu as pltpu

N = 10000
E = 320000
D = 128
H = 128
P = 128
G = 64

INTERPRET = False

ROW_BLK = 1000
N_BLKS = N // ROW_BLK


def _dinv_body(hist_ref, o_ref):
    deg = jnp.sum(hist_ref[...], axis=0) + 1.0
    o_ref[...] = lax.rsqrt(deg)[:, None]


def _tc_dinv(hist):
    return pl.pallas_call(
        _dinv_body,
        out_shape=jax.ShapeDtypeStruct((N, 1), jnp.float32),
        interpret=INTERPRET,
    )(hist)


def _in_body(x_ref, win_ref, bin_ref, w1_ref, dinv_ref, h_ref, y_ref):
    h = jnp.dot(x_ref[...], win_ref[...],
                preferred_element_type=jnp.float32) + bin_ref[...]
    h_ref[...] = h
    y_ref[...] = dinv_ref[...] * jnp.dot(h, w1_ref[...],
                                         preferred_element_type=jnp.float32)


def _tc_in(x, W_in, b_in, W1, dinv):
    return pl.pallas_call(
        _in_body,
        grid=(N_BLKS,),
        in_specs=[
            pl.BlockSpec((ROW_BLK, D), lambda j: (j, 0)),
            pl.BlockSpec((D, H), lambda j: (0, 0)),
            pl.BlockSpec((1, H), lambda j: (0, 0)),
            pl.BlockSpec((H, H), lambda j: (0, 0)),
            pl.BlockSpec((ROW_BLK, 1), lambda j: (j, 0)),
        ],
        out_specs=[
            pl.BlockSpec((ROW_BLK, H), lambda j: (j, 0)),
            pl.BlockSpec((ROW_BLK, H), lambda j: (j, 0)),
        ],
        out_shape=[
            jax.ShapeDtypeStruct((N, H), jnp.float32),
            jax.ShapeDtypeStruct((N, H), jnp.float32),
        ],
        interpret=INTERPRET,
    )(x, W_in, b_in.reshape(1, H), W1, dinv)


def _layer_body(res_ref, acca_ref, accb_ref, y_ref, dinv_ref, b_ref, wn_ref,
                r_ref, yn_ref):
    dinv = dinv_ref[...]
    g = dinv * (acca_ref[...] + accb_ref[...] + y_ref[...]) + b_ref[...]
    r = jnp.maximum(res_ref[...] + g, 0.0)
    r_ref[...] = r
    yn_ref[...] = dinv * jnp.dot(r, wn_ref[...],
                                 preferred_element_type=jnp.float32)


def _tc_layer(res, acc_a, acc_b, y, dinv, b, W_next):
    return pl.pallas_call(
        _layer_body,
        grid=(N_BLKS,),
        in_specs=[
            pl.BlockSpec((ROW_BLK, H), lambda j: (j, 0)),
            pl.BlockSpec((ROW_BLK, H), lambda j: (j, 0)),
            pl.BlockSpec((ROW_BLK, H), lambda j: (j, 0)),
            pl.BlockSpec((ROW_BLK, H), lambda j: (j, 0)),
            pl.BlockSpec((ROW_BLK, 1), lambda j: (j, 0)),
            pl.BlockSpec((1, H), lambda j: (0, 0)),
            pl.BlockSpec((H, H), lambda j: (0, 0)),
        ],
        out_specs=[
            pl.BlockSpec((ROW_BLK, H), lambda j: (j, 0)),
            pl.BlockSpec((ROW_BLK, H), lambda j: (j, 0)),
        ],
        out_shape=[
            jax.ShapeDtypeStruct((N, H), jnp.float32),
            jax.ShapeDtypeStruct((N, H), jnp.float32),
        ],
        interpret=INTERPRET,
    )(res, acc_a, acc_b, y, dinv, b.reshape(1, H), W_next)


def _layer3_body(res_ref, acca_ref, accb_ref, y_ref, dinv_ref, b_ref, r_ref):
    g = dinv_ref[...] * (acca_ref[...] + accb_ref[...] + y_ref[...]) + b_ref[...]
    r_ref[...] = jnp.maximum(res_ref[...] + g, 0.0)


def _tc_layer3(res, acc_a, acc_b, y, dinv, b):
    return pl.pallas_call(
        _layer3_body,
        grid=(N_BLKS,),
        in_specs=[
            pl.BlockSpec((ROW_BLK, H), lambda j: (j, 0)),
            pl.BlockSpec((ROW_BLK, H), lambda j: (j, 0)),
            pl.BlockSpec((ROW_BLK, H), lambda j: (j, 0)),
            pl.BlockSpec((ROW_BLK, H), lambda j: (j, 0)),
            pl.BlockSpec((ROW_BLK, 1), lambda j: (j, 0)),
            pl.BlockSpec((1, H), lambda j: (0, 0)),
        ],
        out_specs=pl.BlockSpec((ROW_BLK, H), lambda j: (j, 0)),
        out_shape=jax.ShapeDtypeStruct((N, H), jnp.float32),
        interpret=INTERPRET,
    )(res, acc_a, acc_b, y, dinv, b.reshape(1, H))


def _pool_body(batch_ref, r1_ref, r2_ref, r3_ref, ps_ref, cnt_ref):
    j = pl.program_id(0)
    b = batch_ref[0, 0, :]
    gids = lax.broadcasted_iota(jnp.int32, (G, ROW_BLK), 0)
    mask = (b[None, :] == gids).astype(jnp.float32)

    @pl.when(j == 0)
    def _():
        ps_ref[...] = jnp.zeros_like(ps_ref)
        cnt_ref[...] = jnp.zeros_like(cnt_ref)

    ps_ref[:, 0:H] += jnp.dot(mask, r1_ref[...],
                              preferred_element_type=jnp.float32)
    ps_ref[:, H:2 * H] += jnp.dot(mask, r2_ref[...],
                                  preferred_element_type=jnp.float32)
    ps_ref[:, 2 * H:3 * H] += jnp.dot(mask, r3_ref[...],
                                      preferred_element_type=jnp.float32)
    cnt_ref[...] += jnp.sum(mask, axis=1, keepdims=True)


def _tc_pool(batch3, r1, r2, r3):
    return pl.pallas_call(
        _pool_body,
        grid=(N_BLKS,),
        in_specs=[
            pl.BlockSpec((1, 1, ROW_BLK), lambda j: (j, 0, 0)),
            pl.BlockSpec((ROW_BLK, H), lambda j: (j, 0)),
            pl.BlockSpec((ROW_BLK, H), lambda j: (j, 0)),
            pl.BlockSpec((ROW_BLK, H), lambda j: (j, 0)),
        ],
        out_specs=[
            pl.BlockSpec((G, 3 * H), lambda j: (0, 0)),
            pl.BlockSpec((G, 1), lambda j: (0, 0)),
        ],
        out_shape=[
            jax.ShapeDtypeStruct((G, 3 * H), jnp.float32),
            jax.ShapeDtypeStruct((G, 1), jnp.float32),
        ],
        interpret=INTERPRET,
    )(batch3, r1, r2, r3)


def _head_body(ps_ref, cnt_ref, wp1_ref, bp1_ref, wp2_ref, bp2_ref, o_ref):
    pooled = ps_ref[...] / jnp.maximum(cnt_ref[...], 1.0)
    t = jnp.maximum(jnp.dot(pooled, wp1_ref[...],
                            preferred_element_type=jnp.float32) + bp1_ref[...],
                    0.0)
    p = jnp.dot(t, wp2_ref[...],
                preferred_element_type=jnp.float32) + bp2_ref[...]
    nrm = jnp.sqrt(jnp.sum(p * p, axis=1, keepdims=True))
    o_ref[...] = p / jnp.maximum(nrm, 1e-12)


def _tc_head(ps, cnt, Wp1, bp1, Wp2, bp2):
    return pl.pallas_call(
        _head_body,
        out_shape=jax.ShapeDtypeStruct((G, P), jnp.float32),
        interpret=INTERPRET,
    )(ps, cnt, Wp1, bp1.reshape(1, H), Wp2, bp2.reshape(1, P))


def _sc_hist(dst):
    h = jax.ops.segment_sum(jnp.ones(E, jnp.float32), dst,
                            num_segments=N).reshape(1, N)
    return jnp.pad(h, ((0, 7), (0, 0)))


def _sc_scatter(y, src, dst):
    acc = jax.ops.segment_sum(y[src], dst, num_segments=N)
    return acc, jnp.zeros_like(acc)


def kernel(x, edge_index, batch, W_in, b_in, W1, b1, W2, b2, W3, b3,
           Wp1, bp1, Wp2, bp2):
    src, dst = edge_index[0], edge_index[1]

    hist = _sc_hist(dst)
    dinv = _tc_dinv(hist)

    h, y1 = _tc_in(x, W_in, b_in, W1, dinv)
    a1, a1b = _sc_scatter(y1, src, dst)
    r1, y2 = _tc_layer(h, a1, a1b, y1, dinv, b1, W2)
    a2, a2b = _sc_scatter(y2, src, dst)
    r2, y3 = _tc_layer(r1, a2, a2b, y2, dinv, b2, W3)
    a3, a3b = _sc_scatter(y3, src, dst)
    r3 = _tc_layer3(r2, a3, a3b, y3, dinv, b3)

    batch3 = batch.reshape(N_BLKS, 1, ROW_BLK)
    ps, cnt = _tc_pool(batch3, r1, r2, r3)
    return _tc_head(ps, cnt, Wp1, bp1, Wp2, bp2)


# speedup vs baseline: 8.9887x; 3.3713x over previous
import functools

import jax
import jax.numpy as jnp
from jax import lax
from jax.experimental import pallas as pl
from jax.experimental.pallas import tpu as pltpu
from jax.experimental.pallas import tpu_sc as plsc

N = 10000
E = 320000
D = 128
H = 128
P = 128
G = 64

INTERPRET = False

ROW_BLK = 1000
N_BLKS = N // ROW_BLK

NC = 2
NS = 16
NW = NC * NS

CHUNK = 128
EPAD = -(-E // (NW * CHUNK)) * (NW * CHUNK)
E_TILE = EPAD // NW
N_CHUNKS = E_TILE // CHUNK
NPAD = 10240
ROWS_TILE = NPAD // NS
EH_TILE = E // NW

_vmesh = plsc.VectorSubcoreMesh(core_axis_name="c", subcore_axis_name="s")

import dataclasses as _dc

_sc_cp = pltpu.CompilerParams()
if "needs_layout_passes" in pltpu.CompilerParams.__dataclass_fields__:
    _sc_cp = _dc.replace(_sc_cp, needs_layout_passes=False)


def _dinv_body(hist_ref, o_ref):
    deg = jnp.sum(hist_ref[...], axis=0) + 1.0
    o_ref[...] = lax.rsqrt(deg)[:, None]


def _tc_dinv(hist):
    return pl.pallas_call(
        _dinv_body,
        out_shape=jax.ShapeDtypeStruct((N, 1), jnp.float32),
        interpret=INTERPRET,
    )(hist)


def _in_body(x_ref, win_ref, bin_ref, w1_ref, dinv_ref, h_ref, y_ref):
    h = jnp.dot(x_ref[...], win_ref[...],
                preferred_element_type=jnp.float32) + bin_ref[...]
    h_ref[...] = h
    y_ref[...] = dinv_ref[...] * jnp.dot(h, w1_ref[...],
                                         preferred_element_type=jnp.float32)


def _tc_in(x, W_in, b_in, W1, dinv):
    return pl.pallas_call(
        _in_body,
        grid=(N_BLKS,),
        in_specs=[
            pl.BlockSpec((ROW_BLK, D), lambda j: (j, 0)),
            pl.BlockSpec((D, H), lambda j: (0, 0)),
            pl.BlockSpec((1, H), lambda j: (0, 0)),
            pl.BlockSpec((H, H), lambda j: (0, 0)),
            pl.BlockSpec((ROW_BLK, 1), lambda j: (j, 0)),
        ],
        out_specs=[
            pl.BlockSpec((ROW_BLK, H), lambda j: (j, 0)),
            pl.BlockSpec((ROW_BLK, H), lambda j: (j, 0)),
        ],
        out_shape=[
            jax.ShapeDtypeStruct((N, H), jnp.float32),
            jax.ShapeDtypeStruct((N, H), jnp.float32),
        ],
        interpret=INTERPRET,
    )(x, W_in, b_in.reshape(1, H), W1, dinv)


def _layer_body(res_ref, acca_ref, accb_ref, y_ref, dinv_ref, b_ref, wn_ref,
                r_ref, yn_ref):
    dinv = dinv_ref[...]
    g = dinv * (acca_ref[...] + accb_ref[...] + y_ref[...]) + b_ref[...]
    r = jnp.maximum(res_ref[...] + g, 0.0)
    r_ref[...] = r
    yn_ref[...] = dinv * jnp.dot(r, wn_ref[...],
                                 preferred_element_type=jnp.float32)


def _tc_layer(res, acc_a, acc_b, y, dinv, b, W_next):
    return pl.pallas_call(
        _layer_body,
        grid=(N_BLKS,),
        in_specs=[
            pl.BlockSpec((ROW_BLK, H), lambda j: (j, 0)),
            pl.BlockSpec((ROW_BLK, H), lambda j: (j, 0)),
            pl.BlockSpec((ROW_BLK, H), lambda j: (j, 0)),
            pl.BlockSpec((ROW_BLK, H), lambda j: (j, 0)),
            pl.BlockSpec((ROW_BLK, 1), lambda j: (j, 0)),
            pl.BlockSpec((1, H), lambda j: (0, 0)),
            pl.BlockSpec((H, H), lambda j: (0, 0)),
        ],
        out_specs=[
            pl.BlockSpec((ROW_BLK, H), lambda j: (j, 0)),
            pl.BlockSpec((ROW_BLK, H), lambda j: (j, 0)),
        ],
        out_shape=[
            jax.ShapeDtypeStruct((N, H), jnp.float32),
            jax.ShapeDtypeStruct((N, H), jnp.float32),
        ],
        interpret=INTERPRET,
    )(res, acc_a, acc_b, y, dinv, b.reshape(1, H), W_next)


def _layer3_body(res_ref, acca_ref, accb_ref, y_ref, dinv_ref, b_ref, r_ref):
    g = dinv_ref[...] * (acca_ref[...] + accb_ref[...] + y_ref[...]) + b_ref[...]
    r_ref[...] = jnp.maximum(res_ref[...] + g, 0.0)


def _tc_layer3(res, acc_a, acc_b, y, dinv, b):
    return pl.pallas_call(
        _layer3_body,
        grid=(N_BLKS,),
        in_specs=[
            pl.BlockSpec((ROW_BLK, H), lambda j: (j, 0)),
            pl.BlockSpec((ROW_BLK, H), lambda j: (j, 0)),
            pl.BlockSpec((ROW_BLK, H), lambda j: (j, 0)),
            pl.BlockSpec((ROW_BLK, H), lambda j: (j, 0)),
            pl.BlockSpec((ROW_BLK, 1), lambda j: (j, 0)),
            pl.BlockSpec((1, H), lambda j: (0, 0)),
        ],
        out_specs=pl.BlockSpec((ROW_BLK, H), lambda j: (j, 0)),
        out_shape=jax.ShapeDtypeStruct((N, H), jnp.float32),
        interpret=INTERPRET,
    )(res, acc_a, acc_b, y, dinv, b.reshape(1, H))


def _pool_body(batch_ref, r1_ref, r2_ref, r3_ref, ps_ref, cnt_ref):
    j = pl.program_id(0)
    b = batch_ref[0, 0, :]
    gids = lax.broadcasted_iota(jnp.int32, (G, ROW_BLK), 0)
    mask = (b[None, :] == gids).astype(jnp.float32)

    @pl.when(j == 0)
    def _():
        ps_ref[...] = jnp.zeros_like(ps_ref)
        cnt_ref[...] = jnp.zeros_like(cnt_ref)

    ps_ref[:, 0:H] += jnp.dot(mask, r1_ref[...],
                              preferred_element_type=jnp.float32)
    ps_ref[:, H:2 * H] += jnp.dot(mask, r2_ref[...],
                                  preferred_element_type=jnp.float32)
    ps_ref[:, 2 * H:3 * H] += jnp.dot(mask, r3_ref[...],
                                      preferred_element_type=jnp.float32)
    cnt_ref[...] += jnp.sum(mask, axis=1, keepdims=True)


def _tc_pool(batch3, r1, r2, r3):
    return pl.pallas_call(
        _pool_body,
        grid=(N_BLKS,),
        in_specs=[
            pl.BlockSpec((1, 1, ROW_BLK), lambda j: (j, 0, 0)),
            pl.BlockSpec((ROW_BLK, H), lambda j: (j, 0)),
            pl.BlockSpec((ROW_BLK, H), lambda j: (j, 0)),
            pl.BlockSpec((ROW_BLK, H), lambda j: (j, 0)),
        ],
        out_specs=[
            pl.BlockSpec((G, 3 * H), lambda j: (0, 0)),
            pl.BlockSpec((G, 1), lambda j: (0, 0)),
        ],
        out_shape=[
            jax.ShapeDtypeStruct((G, 3 * H), jnp.float32),
            jax.ShapeDtypeStruct((G, 1), jnp.float32),
        ],
        interpret=INTERPRET,
    )(batch3, r1, r2, r3)


def _head_body(ps_ref, cnt_ref, wp1_ref, bp1_ref, wp2_ref, bp2_ref, o_ref):
    pooled = ps_ref[...] / jnp.maximum(cnt_ref[...], 1.0)
    t = jnp.maximum(jnp.dot(pooled, wp1_ref[...],
                            preferred_element_type=jnp.float32) + bp1_ref[...],
                    0.0)
    p = jnp.dot(t, wp2_ref[...],
                preferred_element_type=jnp.float32) + bp2_ref[...]
    nrm = jnp.sqrt(jnp.sum(p * p, axis=1, keepdims=True))
    o_ref[...] = p / jnp.maximum(nrm, 1e-12)


def _tc_head(ps, cnt, Wp1, bp1, Wp2, bp2):
    return pl.pallas_call(
        _head_body,
        out_shape=jax.ShapeDtypeStruct((G, P), jnp.float32),
        interpret=INTERPRET,
    )(ps, cnt, Wp1, bp1.reshape(1, H), Wp2, bp2.reshape(1, P))


def _sc_hist(dst):
    @functools.partial(
        pl.kernel,
        out_type=jax.ShapeDtypeStruct((NW, N), jnp.float32),
        mesh=_vmesh,
        compiler_params=_sc_cp,
        scratch_types=[
            pltpu.VMEM((EH_TILE,), jnp.int32),
            pltpu.VMEM((N,), jnp.float32),
        ],
    )
    def k(dst_hbm, out_hbm, idxs, hist):
        wid = lax.axis_index("c") * NS + lax.axis_index("s")
        pltpu.sync_copy(dst_hbm.at[pl.ds(wid * EH_TILE, EH_TILE)], idxs)

        zeros = jnp.zeros((16,), jnp.float32)

        @pl.loop(0, N // 16)
        def _(i):
            hist[pl.ds(i * 16, 16)] = zeros

        ones = jnp.ones((16,), jnp.float32)

        @pl.loop(0, EH_TILE // 16)
        def _(i):
            ii = idxs[pl.ds(i * 16, 16)]
            plsc.addupdate_scatter(hist, [ii], ones)

        pltpu.sync_copy(hist, out_hbm.at[wid])

    return k(dst)


def _sc_scatter(y, srcp, dstp):
    @functools.partial(
        pl.kernel,
        out_type=jax.ShapeDtypeStruct((NC, NPAD, H), jnp.float32),
        mesh=_vmesh,
        compiler_params=_sc_cp,
        scratch_types=[
            pltpu.VMEM((CHUNK,), jnp.int32),
            pltpu.VMEM((CHUNK,), jnp.int32),
            pltpu.VMEM((CHUNK, H), jnp.float32),
            pltpu.VMEM_SHARED((NPAD, H), jnp.float32),
            pltpu.SemaphoreType.DMA,
        ],
    )
    def k(y_hbm, src_hbm, dst_hbm, out_hbm, sidx, didx, rows, acc, sem):
        cid = lax.axis_index("c")
        sid = lax.axis_index("s")

        zeros = jnp.zeros((16,), jnp.float32)

        @pl.loop(0, CHUNK)
        def _(r):
            @pl.loop(0, H // 16)
            def _(q):
                rows[r, pl.ds(q * 16, 16)] = zeros

        @pl.loop(0, ROWS_TILE // CHUNK)
        def _(z):
            pltpu.sync_copy(rows, acc.at[pl.ds(sid * ROWS_TILE + z * CHUNK,
                                               CHUNK)])

        plsc.subcore_barrier()

        base = (cid * NS + sid) * E_TILE

        @pl.loop(0, N_CHUNKS)
        def _(cnk):
            off = base + cnk * CHUNK
            pltpu.sync_copy(src_hbm.at[pl.ds(off, CHUNK)], sidx)
            pltpu.sync_copy(dst_hbm.at[pl.ds(off, CHUNK)], didx)
            pltpu.async_copy(y_hbm.at[sidx], rows, sem).wait()
            pltpu.sync_copy(rows, acc.at[didx], add=True)

        plsc.subcore_barrier()

        pltpu.sync_copy(acc.at[pl.ds(sid * ROWS_TILE, ROWS_TILE)],
                        out_hbm.at[cid, pl.ds(sid * ROWS_TILE, ROWS_TILE)])

    out = k(y, srcp, dstp)
    return out[0, :N], out[1, :N]


def kernel(x, edge_index, batch, W_in, b_in, W1, b1, W2, b2, W3, b3,
           Wp1, bp1, Wp2, bp2):
    src, dst = edge_index[0], edge_index[1]

    srcp = jnp.concatenate([src, jnp.zeros(EPAD - E, jnp.int32)])
    dstp = jnp.concatenate([dst, jnp.full(EPAD - E, NPAD - 1, jnp.int32)])

    hist = _sc_hist(dst)
    dinv = _tc_dinv(hist)

    h, y1 = _tc_in(x, W_in, b_in, W1, dinv)
    a1, a1b = _sc_scatter(y1, srcp, dstp)
    r1, y2 = _tc_layer(h, a1, a1b, y1, dinv, b1, W2)
    a2, a2b = _sc_scatter(y2, srcp, dstp)
    r2, y3 = _tc_layer(r1, a2, a2b, y2, dinv, b2, W3)
    a3, a3b = _sc_scatter(y3, srcp, dstp)
    r3 = _tc_layer3(r2, a3, a3b, y3, dinv, b3)

    batch3 = batch.reshape(N_BLKS, 1, ROW_BLK)
    ps, cnt = _tc_pool(batch3, r1, r2, r3)
    return _tc_head(ps, cnt, Wp1, bp1, Wp2, bp2)
